# Initial kernel scaffold; baseline (speedup 1.0000x reference)
import jax
import jax.numpy as jnp
from jax.experimental import pallas as pl


def kernel(idxs, mask, table, token_weights):
    raise NotImplementedError("write your pallas kernel here")


# speedup vs baseline: 1.2372x; 1.2372x over previous
import functools

import jax
import jax.numpy as jnp
from jax import lax
from jax.experimental import pallas as pl
from jax.experimental.pallas import tpu as pltpu
from jax.experimental.pallas import tpu_sc as plsc

BATCH = 4096
HIST = 200
EMBED = 64
NC = 2
NS = 16
NW = NC * NS
ROWS_W = BATCH // NW
TOK_W = ROWS_W * HIST
C0, C1 = 104, 96
NCHUNK = 13
UNROLL = 8


def _body(idxs_hbm, mask_hbm, table_hbm, tw_hbm, out_hbm,
          idx_all, mask_all, rows0, rows1, tw0, tw1, w_v, out_all,
          sem_r0, sem_r1, sem_t0, sem_t1):
  cid = lax.axis_index("c")
  sid = lax.axis_index("s")
  wid = sid * NC + cid
  row_base = pl.multiple_of(wid * ROWS_W, ROWS_W)
  tok_base = pl.multiple_of(wid * TOK_W, TOK_W)

  pltpu.sync_copy(idxs_hbm.at[pl.ds(row_base, ROWS_W)], idx_all)
  pltpu.sync_copy(mask_hbm.at[pl.ds(tok_base, TOK_W)],
                  mask_all.at[pl.ds(0, TOK_W)])

  rows_bufs = (rows0, rows1)
  tw_bufs = (tw0, tw1)
  sem_r = (sem_r0, sem_r1)
  sem_t = (sem_t0, sem_t1)

  def fire(r, b):
    i0 = idx_all.at[r, pl.ds(0, C0)]
    i1 = idx_all.at[r, pl.ds(C0, C1)]
    pltpu.async_copy(table_hbm.at[i0], rows_bufs[b].at[pl.ds(0, C0)], sem_r[b])
    pltpu.async_copy(table_hbm.at[i1], rows_bufs[b].at[pl.ds(C0, C1)], sem_r[b])
    pltpu.async_copy(tw_hbm.at[i0], tw_bufs[b].at[pl.ds(0, C0)], sem_t[b])
    pltpu.async_copy(tw_hbm.at[i1], tw_bufs[b].at[pl.ds(C0, C1)], sem_t[b])

  def wait(b):
    pltpu.make_async_copy(table_hbm.at[pl.ds(0, HIST)], rows_bufs[b],
                          sem_r[b]).wait()
    pltpu.make_async_copy(tw_hbm.at[pl.ds(0, HIST)],
                          tw_bufs[b].at[pl.ds(0, HIST)], sem_t[b]).wait()

  fire(0, 0)
  fire(1, 1)

  def outer(i, carry):
    for b in range(2):
      r = 2 * i + b
      wait(b)
      rb = pl.multiple_of(r * HIST, 8)
      for c in range(NCHUNK):
        m = mask_all[pl.ds(rb + c * 16, 16)]
        t = tw_bufs[b][pl.ds(c * 16, 16)]
        w_v[pl.ds(c * 16, 16)] = m * m * t

      rows = rows_bufs[b]

      def tok_body(ti, accs):
        a0, a1, a2, a3 = accs
        jb = pl.multiple_of(ti * 16, 16)
        wv = w_v[pl.ds(jb, 16)]
        for u in range(16):
          j = jb + u
          w = wv[u]
          a0 = a0 + rows[j, pl.ds(0, 16)] * w
          a1 = a1 + rows[j, pl.ds(16, 16)] * w
          a2 = a2 + rows[j, pl.ds(32, 16)] * w
          a3 = a3 + rows[j, pl.ds(48, 16)] * w
        return a0, a1, a2, a3

      z = jnp.zeros((16,), jnp.float32)
      a0, a1, a2, a3 = lax.fori_loop(0, HIST // 16, tok_body, (z, z, z, z))
      wv = w_v[pl.ds(192, 16)]
      for u in range(HIST - 16 * (HIST // 16)):
        j = 192 + u
        w = wv[u]
        a0 = a0 + rows[j, pl.ds(0, 16)] * w
        a1 = a1 + rows[j, pl.ds(16, 16)] * w
        a2 = a2 + rows[j, pl.ds(32, 16)] * w
        a3 = a3 + rows[j, pl.ds(48, 16)] * w
      out_all[r, pl.ds(0, 16)] = a0
      out_all[r, pl.ds(16, 16)] = a1
      out_all[r, pl.ds(32, 16)] = a2
      out_all[r, pl.ds(48, 16)] = a3

      @pl.when(r + 2 < ROWS_W)
      def _():
        fire(r + 2, b)
    return carry

  lax.fori_loop(0, ROWS_W // 2, outer, 0)
  pltpu.sync_copy(out_all, out_hbm.at[pl.ds(row_base, ROWS_W)])


@functools.lru_cache(maxsize=1)
def _build():
  return functools.partial(
      pl.kernel,
      out_type=jax.ShapeDtypeStruct((BATCH, EMBED), jnp.float32),
      mesh=plsc.VectorSubcoreMesh(core_axis_name="c", subcore_axis_name="s"),
      scratch_types=[
          pltpu.VMEM((ROWS_W, HIST), jnp.int32),
          pltpu.VMEM((TOK_W + 16,), jnp.float32),
          pltpu.VMEM((HIST, EMBED), jnp.float32),
          pltpu.VMEM((HIST, EMBED), jnp.float32),
          pltpu.VMEM((NCHUNK * 16,), jnp.float32),
          pltpu.VMEM((NCHUNK * 16,), jnp.float32),
          pltpu.VMEM((NCHUNK * 16,), jnp.float32),
          pltpu.VMEM((ROWS_W, EMBED), jnp.float32),
          pltpu.SemaphoreType.DMA,
          pltpu.SemaphoreType.DMA,
          pltpu.SemaphoreType.DMA,
          pltpu.SemaphoreType.DMA,
      ],
      compiler_params=pltpu.CompilerParams(use_tc_tiling_on_sc=False),
  )(_body)


def kernel(idxs, mask, table, token_weights):
  idxs32 = idxs.astype(jnp.int32)
  mask_f = mask.reshape(-1)
  return _build()(idxs32, mask_f, table, token_weights)
